# Initial kernel scaffold; baseline (speedup 1.0000x reference)
import jax
import jax.numpy as jnp
from jax.experimental import pallas as pl


def kernel(z, xyz, cg_xyz, mapping, nbr_list, cg_nbr_list, embed_table, msg_params, cg_params):
    raise NotImplementedError("write your pallas kernel here")


# speedup vs baseline: 1.1924x; 1.1924x over previous
import functools

import jax
import jax.numpy as jnp
from jax.experimental import pallas as pl

N_ATOMS = 10000
N_CG = 1000
N_EDGES = 160000
FEAT = 128
N_RBF = 20
N_CONV = 3
CUTOFF = 5.0
CG_CUTOFF = 20.0

XDIM = N_RBF + 1
XPAD = 24

_ATOM_BLK = 2000
_EDGE_BLK = 3200


def _swish(x):
    return x * jax.nn.sigmoid(x)


def _phi_body(h_ref, w1_ref, b1_ref, w2_ref, b2_ref, out_ref):
    t = jnp.dot(h_ref[...], w1_ref[...], preferred_element_type=jnp.float32)
    t = t + b1_ref[...]
    t = _swish(t)
    out_ref[...] = jnp.dot(t, w2_ref[...], preferred_element_type=jnp.float32) + b2_ref[...]


@jax.jit
def _phi_call(h, w1, b1, w2, b2):
    n = h.shape[0]
    grid = (n // _ATOM_BLK,)
    return pl.pallas_call(
        _phi_body,
        grid=grid,
        in_specs=[
            pl.BlockSpec((_ATOM_BLK, FEAT), lambda i: (i, 0)),
            pl.BlockSpec((FEAT, FEAT), lambda i: (0, 0)),
            pl.BlockSpec((1, FEAT), lambda i: (0, 0)),
            pl.BlockSpec((FEAT, FEAT), lambda i: (0, 0)),
            pl.BlockSpec((1, FEAT), lambda i: (0, 0)),
        ],
        out_specs=pl.BlockSpec((_ATOM_BLK, FEAT), lambda i: (i, 0)),
        out_shape=jax.ShapeDtypeStruct((n, FEAT), jnp.float32),
    )(h, w1, b1, w2, b2)


def _edge_s0_body(phi_g_ref, x_ref, waug_ref, out_ref):
    w = jnp.dot(x_ref[...], waug_ref[...], preferred_element_type=jnp.float32)
    out_ref[...] = phi_g_ref[...] * w


@jax.jit
def _edge_s0_call(phi_g, x_e, waug):
    n = phi_g.shape[0]
    blk = _EDGE_BLK if n % _EDGE_BLK == 0 else _ATOM_BLK
    grid = (n // blk,)
    return pl.pallas_call(
        _edge_s0_body,
        grid=grid,
        in_specs=[
            pl.BlockSpec((blk, FEAT), lambda i: (i, 0)),
            pl.BlockSpec((blk, XPAD), lambda i: (i, 0)),
            pl.BlockSpec((XPAD, FEAT), lambda i: (0, 0)),
        ],
        out_specs=pl.BlockSpec((blk, FEAT), lambda i: (i, 0)),
        out_shape=jax.ShapeDtypeStruct((n, FEAT), jnp.float32),
    )(phi_g, x_e, waug)


def _contract_body(h_ref, w1_ref, b1_ref, w2_ref, b2_ref, x_ref, waug_ref, out_ref):
    t = jnp.dot(h_ref[...], w1_ref[...], preferred_element_type=jnp.float32)
    t = t + b1_ref[...]
    t = _swish(t)
    phi = jnp.dot(t, w2_ref[...], preferred_element_type=jnp.float32) + b2_ref[...]
    w = jnp.dot(x_ref[...], waug_ref[...], preferred_element_type=jnp.float32)
    out_ref[...] = phi * w


@jax.jit
def _contract_call(h, w1, b1, w2, b2, x_c, waug):
    n = h.shape[0]
    grid = (n // _ATOM_BLK,)
    return pl.pallas_call(
        _contract_body,
        grid=grid,
        in_specs=[
            pl.BlockSpec((_ATOM_BLK, FEAT), lambda i: (i, 0)),
            pl.BlockSpec((FEAT, FEAT), lambda i: (0, 0)),
            pl.BlockSpec((1, FEAT), lambda i: (0, 0)),
            pl.BlockSpec((FEAT, FEAT), lambda i: (0, 0)),
            pl.BlockSpec((1, FEAT), lambda i: (0, 0)),
            pl.BlockSpec((_ATOM_BLK, XPAD), lambda i: (i, 0)),
            pl.BlockSpec((XPAD, FEAT), lambda i: (0, 0)),
        ],
        out_specs=pl.BlockSpec((_ATOM_BLK, FEAT), lambda i: (i, 0)),
        out_shape=jax.ShapeDtypeStruct((n, FEAT), jnp.float32),
    )(h, w1, b1, w2, b2, x_c, waug)


def _dist_features(r, cutoff):
    d = jnp.sqrt(jnp.sum(r * r, axis=-1) + 1e-12)
    n = jnp.arange(1, N_RBF + 1, dtype=jnp.float32)
    rbf = jnp.sin(n[None, :] * (jnp.pi / cutoff) * d[:, None]) / d[:, None]
    env = jnp.where(d < cutoff, 0.5 * (jnp.cos(jnp.pi * d / cutoff) + 1.0), 0.0)
    x = jnp.concatenate([rbf * env[:, None], env[:, None]], axis=1)
    return jnp.pad(x, ((0, 0), (0, XPAD - XDIM)))


def _aug_filter(p):
    w = jnp.concatenate([p['Wd'][:, :FEAT], p['bd'][None, :FEAT]], axis=0)
    return jnp.pad(w, ((0, XPAD - XDIM), (0, 0)))


def kernel(z, xyz, cg_xyz, mapping, nbr_list, cg_nbr_list, embed_table, msg_params, cg_params):
    h = embed_table[z]
    nbr0 = nbr_list[:, 0]
    nbr1 = nbr_list[:, 1]

    r_ij = xyz[nbr1] - xyz[nbr0]
    x_e = _dist_features(r_ij, CUTOFF)
    r_iI = xyz - cg_xyz[mapping]
    x_c = _dist_features(r_iI, CG_CUTOFF)

    H = jnp.zeros((N_CG, FEAT), dtype=jnp.float32)
    for i in range(N_CONV):
        p = msg_params[i]
        phi = _phi_call(h, p['W1'], p['b1'][None, :], p['W2'][:, :FEAT], p['b2'][None, :FEAT])
        s0 = _edge_s0_call(phi[nbr1], x_e, _aug_filter(p))
        ds = jax.ops.segment_sum(s0, nbr0, num_segments=N_ATOMS)
        h = h + ds
        if i == 0:
            cnt = jax.ops.segment_sum(jnp.ones((N_ATOMS,), jnp.float32), mapping, num_segments=N_CG)
            H = jax.ops.segment_sum(h, mapping, num_segments=N_CG) / jnp.clip(cnt, 1.0)[:, None]
        q = cg_params[i]
        sc = _contract_call(h, q['W1'], q['b1'][None, :], q['W2'][:, :FEAT], q['b2'][None, :FEAT],
                            x_c, _aug_filter(q))
        H = H + jax.ops.segment_sum(sc, mapping, num_segments=N_CG)
    return (H, h)


# speedup vs baseline: 1.9999x; 1.6772x over previous
import functools

import jax
import jax.numpy as jnp
from jax import lax
from jax.experimental import pallas as pl
from jax.experimental.pallas import tpu as pltpu
from jax.experimental.pallas import tpu_sc as plsc

N_ATOMS = 10000
N_CG = 1000
N_EDGES = 160000
FEAT = 128
N_RBF = 20
N_CONV = 3
CUTOFF = 5.0
CG_CUTOFF = 20.0

XDIM = N_RBF + 1
XPAD = 24

_ATOM_BLK = 2000
_EDGE_BLK = 3200


def _swish(x):
    return x * jax.nn.sigmoid(x)


def _phi_body(h_ref, w1_ref, b1_ref, w2_ref, b2_ref, out_ref):
    t = jnp.dot(h_ref[...], w1_ref[...], preferred_element_type=jnp.float32)
    t = t + b1_ref[...]
    t = _swish(t)
    out_ref[...] = jnp.dot(t, w2_ref[...], preferred_element_type=jnp.float32) + b2_ref[...]


@jax.jit
def _phi_call(h, w1, b1, w2, b2):
    n = h.shape[0]
    grid = (n // _ATOM_BLK,)
    return pl.pallas_call(
        _phi_body,
        grid=grid,
        in_specs=[
            pl.BlockSpec((_ATOM_BLK, FEAT), lambda i: (i, 0)),
            pl.BlockSpec((FEAT, FEAT), lambda i: (0, 0)),
            pl.BlockSpec((1, FEAT), lambda i: (0, 0)),
            pl.BlockSpec((FEAT, FEAT), lambda i: (0, 0)),
            pl.BlockSpec((1, FEAT), lambda i: (0, 0)),
        ],
        out_specs=pl.BlockSpec((_ATOM_BLK, FEAT), lambda i: (i, 0)),
        out_shape=jax.ShapeDtypeStruct((n, FEAT), jnp.float32),
    )(h, w1, b1, w2, b2)


def _edge_s0_body(phi_g_ref, x_ref, waug_ref, out_ref):
    w = jnp.dot(x_ref[...], waug_ref[...], preferred_element_type=jnp.float32)
    out_ref[...] = phi_g_ref[...] * w


@jax.jit
def _edge_s0_call(phi_g, x_e, waug):
    n = phi_g.shape[0]
    blk = _EDGE_BLK if n % _EDGE_BLK == 0 else _ATOM_BLK
    grid = (n // blk,)
    return pl.pallas_call(
        _edge_s0_body,
        grid=grid,
        in_specs=[
            pl.BlockSpec((blk, FEAT), lambda i: (i, 0)),
            pl.BlockSpec((blk, XPAD), lambda i: (i, 0)),
            pl.BlockSpec((XPAD, FEAT), lambda i: (0, 0)),
        ],
        out_specs=pl.BlockSpec((blk, FEAT), lambda i: (i, 0)),
        out_shape=jax.ShapeDtypeStruct((n, FEAT), jnp.float32),
    )(phi_g, x_e, waug)


def _contract_body(h_ref, p0_ref, p1_ref, w1_ref, b1_ref, w2_ref, b2_ref,
                   x_ref, waug_ref, hnew_ref, out_ref):
    hn = h_ref[...] + p0_ref[0] + p1_ref[0]
    hnew_ref[...] = hn
    t = jnp.dot(hn, w1_ref[...], preferred_element_type=jnp.float32)
    t = t + b1_ref[...]
    t = _swish(t)
    phi = jnp.dot(t, w2_ref[...], preferred_element_type=jnp.float32) + b2_ref[...]
    w = jnp.dot(x_ref[...], waug_ref[...], preferred_element_type=jnp.float32)
    out_ref[...] = phi * w


@jax.jit
def _contract_call(h, dsp, w1, b1, w2, b2, x_c, waug):
    n = h.shape[0]
    grid = (n // _ATOM_BLK,)
    return pl.pallas_call(
        _contract_body,
        grid=grid,
        in_specs=[
            pl.BlockSpec((_ATOM_BLK, FEAT), lambda i: (i, 0)),
            pl.BlockSpec((1, _ATOM_BLK, FEAT), lambda i: (0, i, 0)),
            pl.BlockSpec((1, _ATOM_BLK, FEAT), lambda i: (1, i, 0)),
            pl.BlockSpec((FEAT, FEAT), lambda i: (0, 0)),
            pl.BlockSpec((1, FEAT), lambda i: (0, 0)),
            pl.BlockSpec((FEAT, FEAT), lambda i: (0, 0)),
            pl.BlockSpec((1, FEAT), lambda i: (0, 0)),
            pl.BlockSpec((_ATOM_BLK, XPAD), lambda i: (i, 0)),
            pl.BlockSpec((XPAD, FEAT), lambda i: (0, 0)),
        ],
        out_specs=[
            pl.BlockSpec((_ATOM_BLK, FEAT), lambda i: (i, 0)),
            pl.BlockSpec((_ATOM_BLK, FEAT), lambda i: (i, 0)),
        ],
        out_shape=[
            jax.ShapeDtypeStruct((n, FEAT), jnp.float32),
            jax.ShapeDtypeStruct((n, FEAT), jnp.float32),
        ],
    )(h, dsp, dsp, w1, b1, w2, b2, x_c, waug)


_NC = 2
_NS = 16
_NW = _NC * _NS
_EC = 128
_ECHUNKS = N_EDGES // _EC
_ROWS_PER_TILE = 624


def _edge_sc_body(phi_hbm, w_hbm, n0_hbm, n1_hbm, out_hbm,
                  acc, idx0_v, idx1_v, w_v, g_v):
    c = lax.axis_index("c")
    s = lax.axis_index("s")
    wid = s * _NC + c

    def _zfill(r, carry):
        for k in range(8):
            w_v[r, pl.ds(k * 16, 16)] = jnp.zeros((16,), jnp.float32)
        return carry
    lax.fori_loop(0, _EC, _zfill, 0)
    base_r = s * _ROWS_PER_TILE
    for k in range(4):
        pltpu.sync_copy(w_v, acc.at[pl.ds(base_r + k * _EC, _EC), :])
    pltpu.sync_copy(w_v.at[pl.ds(0, _ROWS_PER_TILE - 4 * _EC), :],
                    acc.at[pl.ds(base_r + 4 * _EC, _ROWS_PER_TILE - 4 * _EC), :])

    @pl.when(s == _NS - 1)
    def _zero_tail():
        pltpu.sync_copy(w_v.at[pl.ds(0, N_ATOMS - _NS * _ROWS_PER_TILE), :],
                        acc.at[pl.ds(_NS * _ROWS_PER_TILE, N_ATOMS - _NS * _ROWS_PER_TILE), :])
    plsc.subcore_barrier()

    nch = jnp.where(wid < _ECHUNKS - (_ECHUNKS // _NW) * _NW,
                    _ECHUNKS // _NW + 1, _ECHUNKS // _NW)

    def _chunk(j, carry):
        base = (j * _NW + wid) * _EC
        pltpu.sync_copy(n0_hbm.at[pl.ds(base, _EC)], idx0_v)
        pltpu.sync_copy(n1_hbm.at[pl.ds(base, _EC)], idx1_v)
        pltpu.sync_copy(w_hbm.at[pl.ds(base, _EC), :], w_v)
        pltpu.sync_copy(phi_hbm.at[idx1_v], g_v)

        def _mul(r, inner):
            for k in range(8):
                sl = pl.ds(k * 16, 16)
                g_v[r, sl] = g_v[r, sl] * w_v[r, sl]
            return inner
        lax.fori_loop(0, _EC, _mul, 0)
        pltpu.sync_copy(g_v, acc.at[idx0_v], add=True)
        return carry
    lax.fori_loop(0, nch, _chunk, 0)
    plsc.subcore_barrier()
    pltpu.sync_copy(acc.at[pl.ds(base_r, _ROWS_PER_TILE), :],
                    out_hbm.at[c, pl.ds(base_r, _ROWS_PER_TILE), :])

    @pl.when(s == _NS - 1)
    def _flush_tail():
        pltpu.sync_copy(acc.at[pl.ds(_NS * _ROWS_PER_TILE, N_ATOMS - _NS * _ROWS_PER_TILE), :],
                        out_hbm.at[c, pl.ds(_NS * _ROWS_PER_TILE, N_ATOMS - _NS * _ROWS_PER_TILE), :])


@jax.jit
def _edge_sc_call(phi, w_e, nbr0, nbr1):
    return pl.kernel(
        _edge_sc_body,
        out_type=jax.ShapeDtypeStruct((_NC, N_ATOMS, FEAT), jnp.float32),
        mesh=plsc.VectorSubcoreMesh(core_axis_name="c", subcore_axis_name="s"),
        scratch_types=[
            pltpu.VMEM_SHARED((N_ATOMS, FEAT), jnp.float32),
            pltpu.VMEM((_EC,), jnp.int32),
            pltpu.VMEM((_EC,), jnp.int32),
            pltpu.VMEM((_EC, FEAT), jnp.float32),
            pltpu.VMEM((_EC, FEAT), jnp.float32),
        ],
    )(phi, w_e, nbr0, nbr1)


def _w_body(x_ref, waug_ref, out_ref):
    out_ref[...] = jnp.dot(x_ref[...], waug_ref[...], preferred_element_type=jnp.float32)


@jax.jit
def _w_call(x_e, waug):
    n = x_e.shape[0]
    blk = _EDGE_BLK if n % _EDGE_BLK == 0 else _ATOM_BLK
    return pl.pallas_call(
        _w_body,
        grid=(n // blk,),
        in_specs=[
            pl.BlockSpec((blk, XPAD), lambda i: (i, 0)),
            pl.BlockSpec((XPAD, FEAT), lambda i: (0, 0)),
        ],
        out_specs=pl.BlockSpec((blk, FEAT), lambda i: (i, 0)),
        out_shape=jax.ShapeDtypeStruct((n, FEAT), jnp.float32),
    )(x_e, waug)


def _dist_features(r, cutoff):
    d = jnp.sqrt(jnp.sum(r * r, axis=-1) + 1e-12)
    n = jnp.arange(1, N_RBF + 1, dtype=jnp.float32)
    rbf = jnp.sin(n[None, :] * (jnp.pi / cutoff) * d[:, None]) / d[:, None]
    env = jnp.where(d < cutoff, 0.5 * (jnp.cos(jnp.pi * d / cutoff) + 1.0), 0.0)
    x = jnp.concatenate([rbf * env[:, None], env[:, None]], axis=1)
    return jnp.pad(x, ((0, 0), (0, XPAD - XDIM)))


def _aug_filter(p):
    w = jnp.concatenate([p['Wd'][:, :FEAT], p['bd'][None, :FEAT]], axis=0)
    return jnp.pad(w, ((0, XPAD - XDIM), (0, 0)))


def kernel(z, xyz, cg_xyz, mapping, nbr_list, cg_nbr_list, embed_table, msg_params, cg_params):
    h = embed_table[z]
    nbr0 = nbr_list[:, 0]
    nbr1 = nbr_list[:, 1]

    r_ij = xyz[nbr1] - xyz[nbr0]
    x_e = _dist_features(r_ij, CUTOFF)
    r_iI = xyz - cg_xyz[mapping]
    x_c = _dist_features(r_iI, CG_CUTOFF)

    nbr0 = jnp.asarray(nbr0, jnp.int32)
    nbr1 = jnp.asarray(nbr1, jnp.int32)
    H = jnp.zeros((N_CG, FEAT), dtype=jnp.float32)
    for i in range(N_CONV):
        p = msg_params[i]
        phi = _phi_call(h, p['W1'], p['b1'][None, :], p['W2'][:, :FEAT], p['b2'][None, :FEAT])
        w_e = _w_call(x_e, _aug_filter(p))
        dsp = _edge_sc_call(phi, w_e, nbr0, nbr1)
        q = cg_params[i]
        h, sc = _contract_call(h, dsp, q['W1'], q['b1'][None, :], q['W2'][:, :FEAT],
                               q['b2'][None, :FEAT], x_c, _aug_filter(q))
        if i == 0:
            cnt = jax.ops.segment_sum(jnp.ones((N_ATOMS,), jnp.float32), mapping, num_segments=N_CG)
            H = jax.ops.segment_sum(h, mapping, num_segments=N_CG) / jnp.clip(cnt, 1.0)[:, None]
        H = H + jax.ops.segment_sum(sc, mapping, num_segments=N_CG)
    return (H, h)


# speedup vs baseline: 2.3195x; 1.1598x over previous
import functools

import jax
import jax.numpy as jnp
from jax import lax
from jax.experimental import pallas as pl
from jax.experimental.pallas import tpu as pltpu
from jax.experimental.pallas import tpu_sc as plsc

N_ATOMS = 10000
N_CG = 1000
N_EDGES = 160000
FEAT = 128
N_RBF = 20
N_CONV = 3
CUTOFF = 5.0
CG_CUTOFF = 20.0

XDIM = N_RBF + 1
XPAD = 24

_ATOM_BLK = 2000
_EDGE_BLK = 3200


def _swish(x):
    return x * jax.nn.sigmoid(x)


def _phi_body(h_ref, w1_ref, b1_ref, w2_ref, b2_ref, out_ref):
    t = jnp.dot(h_ref[...], w1_ref[...], preferred_element_type=jnp.float32)
    t = t + b1_ref[...]
    t = _swish(t)
    out_ref[...] = jnp.dot(t, w2_ref[...], preferred_element_type=jnp.float32) + b2_ref[...]


@jax.jit
def _phi_call(h, w1, b1, w2, b2):
    n = h.shape[0]
    grid = (n // _ATOM_BLK,)
    return pl.pallas_call(
        _phi_body,
        grid=grid,
        in_specs=[
            pl.BlockSpec((_ATOM_BLK, FEAT), lambda i: (i, 0)),
            pl.BlockSpec((FEAT, FEAT), lambda i: (0, 0)),
            pl.BlockSpec((1, FEAT), lambda i: (0, 0)),
            pl.BlockSpec((FEAT, FEAT), lambda i: (0, 0)),
            pl.BlockSpec((1, FEAT), lambda i: (0, 0)),
        ],
        out_specs=pl.BlockSpec((_ATOM_BLK, FEAT), lambda i: (i, 0)),
        out_shape=jax.ShapeDtypeStruct((n, FEAT), jnp.float32),
    )(h, w1, b1, w2, b2)


def _edge_s0_body(phi_g_ref, x_ref, waug_ref, out_ref):
    w = jnp.dot(x_ref[...], waug_ref[...], preferred_element_type=jnp.float32)
    out_ref[...] = phi_g_ref[...] * w


@jax.jit
def _edge_s0_call(phi_g, x_e, waug):
    n = phi_g.shape[0]
    blk = _EDGE_BLK if n % _EDGE_BLK == 0 else _ATOM_BLK
    grid = (n // blk,)
    return pl.pallas_call(
        _edge_s0_body,
        grid=grid,
        in_specs=[
            pl.BlockSpec((blk, FEAT), lambda i: (i, 0)),
            pl.BlockSpec((blk, XPAD), lambda i: (i, 0)),
            pl.BlockSpec((XPAD, FEAT), lambda i: (0, 0)),
        ],
        out_specs=pl.BlockSpec((blk, FEAT), lambda i: (i, 0)),
        out_shape=jax.ShapeDtypeStruct((n, FEAT), jnp.float32),
    )(phi_g, x_e, waug)


def _contract_body(h_ref, p0_ref, p1_ref, w1_ref, b1_ref, w2_ref, b2_ref,
                   x_ref, waug_ref, hnew_ref, out_ref):
    hn = h_ref[...] + p0_ref[0] + p1_ref[0]
    hnew_ref[...] = hn
    t = jnp.dot(hn, w1_ref[...], preferred_element_type=jnp.float32)
    t = t + b1_ref[...]
    t = _swish(t)
    phi = jnp.dot(t, w2_ref[...], preferred_element_type=jnp.float32) + b2_ref[...]
    w = jnp.dot(x_ref[...], waug_ref[...], preferred_element_type=jnp.float32)
    out_ref[...] = phi * w


@jax.jit
def _contract_call(h, dsp, w1, b1, w2, b2, x_c, waug):
    n = h.shape[0]
    grid = (n // _ATOM_BLK,)
    return pl.pallas_call(
        _contract_body,
        grid=grid,
        in_specs=[
            pl.BlockSpec((_ATOM_BLK, FEAT), lambda i: (i, 0)),
            pl.BlockSpec((1, _ATOM_BLK, FEAT), lambda i: (0, i, 0)),
            pl.BlockSpec((1, _ATOM_BLK, FEAT), lambda i: (1, i, 0)),
            pl.BlockSpec((FEAT, FEAT), lambda i: (0, 0)),
            pl.BlockSpec((1, FEAT), lambda i: (0, 0)),
            pl.BlockSpec((FEAT, FEAT), lambda i: (0, 0)),
            pl.BlockSpec((1, FEAT), lambda i: (0, 0)),
            pl.BlockSpec((_ATOM_BLK, XPAD), lambda i: (i, 0)),
            pl.BlockSpec((XPAD, FEAT), lambda i: (0, 0)),
        ],
        out_specs=[
            pl.BlockSpec((_ATOM_BLK, FEAT), lambda i: (i, 0)),
            pl.BlockSpec((_ATOM_BLK, FEAT), lambda i: (i, 0)),
        ],
        out_shape=[
            jax.ShapeDtypeStruct((n, FEAT), jnp.float32),
            jax.ShapeDtypeStruct((n, FEAT), jnp.float32),
        ],
    )(h, dsp, dsp, w1, b1, w2, b2, x_c, waug)


_NC = 2
_NS = 16
_NW = _NC * _NS
_EC = 128
_ECHUNKS = N_EDGES // _EC
_ROWS_PER_TILE = 624


def _edge_sc_body(phi_hbm, w_hbm, n0_hbm, n1_hbm, out_hbm,
                  acc, idx0_v, idx1_v, w_v, g_v):
    c = lax.axis_index("c")
    s = lax.axis_index("s")
    wid = s * _NC + c

    def _zfill(r, carry):
        for k in range(8):
            w_v[r, pl.ds(k * 16, 16)] = jnp.zeros((16,), jnp.float32)
        return carry
    lax.fori_loop(0, _EC, _zfill, 0)
    base_r = s * _ROWS_PER_TILE
    for k in range(4):
        pltpu.sync_copy(w_v, acc.at[pl.ds(base_r + k * _EC, _EC), :])
    pltpu.sync_copy(w_v.at[pl.ds(0, _ROWS_PER_TILE - 4 * _EC), :],
                    acc.at[pl.ds(base_r + 4 * _EC, _ROWS_PER_TILE - 4 * _EC), :])

    @pl.when(s == _NS - 1)
    def _zero_tail():
        pltpu.sync_copy(w_v.at[pl.ds(0, N_ATOMS - _NS * _ROWS_PER_TILE), :],
                        acc.at[pl.ds(_NS * _ROWS_PER_TILE, N_ATOMS - _NS * _ROWS_PER_TILE), :])
    plsc.subcore_barrier()

    nch = jnp.where(wid < _ECHUNKS - (_ECHUNKS // _NW) * _NW,
                    _ECHUNKS // _NW + 1, _ECHUNKS // _NW)

    def _chunk(j, carry):
        base = (j * _NW + wid) * _EC
        pltpu.sync_copy(n0_hbm.at[pl.ds(base, _EC)], idx0_v)
        pltpu.sync_copy(n1_hbm.at[pl.ds(base, _EC)], idx1_v)
        pltpu.sync_copy(w_hbm.at[pl.ds(base, _EC), :], w_v)
        pltpu.sync_copy(phi_hbm.at[idx1_v], g_v)

        def _mul(r, inner):
            for k in range(8):
                sl = pl.ds(k * 16, 16)
                g_v[r, sl] = g_v[r, sl] * w_v[r, sl]
            return inner
        lax.fori_loop(0, _EC, _mul, 0)
        pltpu.sync_copy(g_v, acc.at[idx0_v], add=True)
        return carry
    lax.fori_loop(0, nch, _chunk, 0)
    plsc.subcore_barrier()
    pltpu.sync_copy(acc.at[pl.ds(base_r, _ROWS_PER_TILE), :],
                    out_hbm.at[c, pl.ds(base_r, _ROWS_PER_TILE), :])

    @pl.when(s == _NS - 1)
    def _flush_tail():
        pltpu.sync_copy(acc.at[pl.ds(_NS * _ROWS_PER_TILE, N_ATOMS - _NS * _ROWS_PER_TILE), :],
                        out_hbm.at[c, pl.ds(_NS * _ROWS_PER_TILE, N_ATOMS - _NS * _ROWS_PER_TILE), :])


@jax.jit
def _edge_sc_call(phi, w_e, nbr0, nbr1):
    return pl.kernel(
        _edge_sc_body,
        out_type=jax.ShapeDtypeStruct((_NC, N_ATOMS, FEAT), jnp.float32),
        mesh=plsc.VectorSubcoreMesh(core_axis_name="c", subcore_axis_name="s"),
        scratch_types=[
            pltpu.VMEM_SHARED((N_ATOMS, FEAT), jnp.float32),
            pltpu.VMEM((_EC,), jnp.int32),
            pltpu.VMEM((_EC,), jnp.int32),
            pltpu.VMEM((_EC, FEAT), jnp.float32),
            pltpu.VMEM((_EC, FEAT), jnp.float32),
        ],
    )(phi, w_e, nbr0, nbr1)


_APAD = 10240
_CGPAD = 1024
_CROWS = _APAD // _NW


def _cg_zero_flush(c, s, accs, buf, outs):
    def _zfill(r, carry):
        for k in range(8):
            buf[r, pl.ds(k * 16, 16)] = jnp.zeros((16,), jnp.float32)
        return carry
    lax.fori_loop(0, 64, _zfill, 0)
    for acc in accs:
        w = acc.shape[1]
        pltpu.sync_copy(buf.at[pl.ds(0, 64), pl.ds(0, w)], acc.at[pl.ds(s * 64, 64), :])


def _contract_sc_body(sc_hbm, map_hbm, out_hbm, acc, idxA, idxB, buf):
    c = lax.axis_index("c")
    s = lax.axis_index("s")
    wid = s * _NC + c
    _cg_zero_flush(c, s, [acc], buf, None)
    plsc.subcore_barrier()
    base = wid * _CROWS
    for off in (0, 128):
        pltpu.sync_copy(map_hbm.at[pl.ds(base + off, 128)], idxA)
        pltpu.sync_copy(sc_hbm.at[pl.ds(base + off, 128), :], buf)
        pltpu.sync_copy(buf, acc.at[idxA], add=True)
    pltpu.sync_copy(map_hbm.at[pl.ds(base + 256, 64)], idxB)
    pltpu.sync_copy(sc_hbm.at[pl.ds(base + 256, 64), :], buf.at[pl.ds(0, 64), :])
    pltpu.sync_copy(buf.at[pl.ds(0, 64), :], acc.at[idxB], add=True)
    plsc.subcore_barrier()
    pltpu.sync_copy(acc.at[pl.ds(s * 64, 64), :], out_hbm.at[c, pl.ds(s * 64, 64), :])


@jax.jit
def _contract_sc_call(sc_pad, map_pad):
    return pl.kernel(
        _contract_sc_body,
        out_type=jax.ShapeDtypeStruct((_NC, _CGPAD, FEAT), jnp.float32),
        mesh=plsc.VectorSubcoreMesh(core_axis_name="c", subcore_axis_name="s"),
        scratch_types=[
            pltpu.VMEM_SHARED((_CGPAD, FEAT), jnp.float32),
            pltpu.VMEM((128,), jnp.int32),
            pltpu.VMEM((64,), jnp.int32),
            pltpu.VMEM((128, FEAT), jnp.float32),
        ],
    )(sc_pad, map_pad)


def _contract0_sc_body(sc_hbm, h_hbm, map_hbm, out_hbm, outsh_hbm, outcnt_hbm,
                       acc, accsh, acccnt, idxA, idxB, buf, ones_v):
    c = lax.axis_index("c")
    s = lax.axis_index("s")
    wid = s * _NC + c

    def _ofill(r, carry):
        for k in range(8):
            ones_v[r, pl.ds(k * 16, 16)] = jnp.full((16,), 1.0, jnp.float32)
        return carry
    lax.fori_loop(0, 128, _ofill, 0)
    _cg_zero_flush(c, s, [acc, accsh, acccnt], buf, None)
    plsc.subcore_barrier()
    base = wid * _CROWS
    for off in (0, 128):
        pltpu.sync_copy(map_hbm.at[pl.ds(base + off, 128)], idxA)
        pltpu.sync_copy(sc_hbm.at[pl.ds(base + off, 128), :], buf)
        pltpu.sync_copy(buf, acc.at[idxA], add=True)
        pltpu.sync_copy(h_hbm.at[pl.ds(base + off, 128), :], buf)
        pltpu.sync_copy(buf, accsh.at[idxA], add=True)
        pltpu.sync_copy(ones_v, acccnt.at[idxA], add=True)
    pltpu.sync_copy(map_hbm.at[pl.ds(base + 256, 64)], idxB)
    pltpu.sync_copy(sc_hbm.at[pl.ds(base + 256, 64), :], buf.at[pl.ds(0, 64), :])
    pltpu.sync_copy(buf.at[pl.ds(0, 64), :], acc.at[idxB], add=True)
    pltpu.sync_copy(h_hbm.at[pl.ds(base + 256, 64), :], buf.at[pl.ds(0, 64), :])
    pltpu.sync_copy(buf.at[pl.ds(0, 64), :], accsh.at[idxB], add=True)
    pltpu.sync_copy(ones_v.at[pl.ds(0, 64), :], acccnt.at[idxB], add=True)
    plsc.subcore_barrier()
    pltpu.sync_copy(acc.at[pl.ds(s * 64, 64), :], out_hbm.at[c, pl.ds(s * 64, 64), :])
    pltpu.sync_copy(accsh.at[pl.ds(s * 64, 64), :], outsh_hbm.at[c, pl.ds(s * 64, 64), :])
    pltpu.sync_copy(acccnt.at[pl.ds(s * 64, 64), :], outcnt_hbm.at[c, pl.ds(s * 64, 64), :])


@jax.jit
def _contract0_sc_call(sc_pad, h_pad, map_pad):
    return pl.kernel(
        _contract0_sc_body,
        out_type=(
            jax.ShapeDtypeStruct((_NC, _CGPAD, FEAT), jnp.float32),
            jax.ShapeDtypeStruct((_NC, _CGPAD, FEAT), jnp.float32),
            jax.ShapeDtypeStruct((_NC, _CGPAD, FEAT), jnp.float32),
        ),
        mesh=plsc.VectorSubcoreMesh(core_axis_name="c", subcore_axis_name="s"),
        scratch_types=[
            pltpu.VMEM_SHARED((_CGPAD, FEAT), jnp.float32),
            pltpu.VMEM_SHARED((_CGPAD, FEAT), jnp.float32),
            pltpu.VMEM_SHARED((_CGPAD, FEAT), jnp.float32),
            pltpu.VMEM((128,), jnp.int32),
            pltpu.VMEM((64,), jnp.int32),
            pltpu.VMEM((128, FEAT), jnp.float32),
            pltpu.VMEM((128, FEAT), jnp.float32),
        ],
    )(sc_pad, h_pad, map_pad)


def _assemble_body(sh_ref, cnt_ref, d0_ref, d1_ref, d2_ref, out_ref):
    sh = sh_ref[0] + sh_ref[1]
    cnt = cnt_ref[0][:, :1] + cnt_ref[1][:, :1]
    out_ref[...] = (sh / jnp.clip(cnt, 1.0)
                    + d0_ref[0] + d0_ref[1]
                    + d1_ref[0] + d1_ref[1]
                    + d2_ref[0] + d2_ref[1])


@jax.jit
def _assemble_call(shp, cntp, d0p, d1p, d2p):
    full = lambda shape: pl.BlockSpec(shape, lambda: (0, 0, 0))
    return pl.pallas_call(
        _assemble_body,
        in_specs=[
            full((_NC, _CGPAD, FEAT)),
            full((_NC, _CGPAD, FEAT)),
            full((_NC, _CGPAD, FEAT)),
            full((_NC, _CGPAD, FEAT)),
            full((_NC, _CGPAD, FEAT)),
        ],
        out_specs=pl.BlockSpec((_CGPAD, FEAT), lambda: (0, 0)),
        out_shape=jax.ShapeDtypeStruct((_CGPAD, FEAT), jnp.float32),
    )(shp, cntp, d0p, d1p, d2p)


def _w_body(x_ref, waug_ref, out_ref):
    out_ref[...] = jnp.dot(x_ref[...], waug_ref[...], preferred_element_type=jnp.float32)


@jax.jit
def _w_call(x_e, waug):
    n = x_e.shape[0]
    blk = _EDGE_BLK if n % _EDGE_BLK == 0 else _ATOM_BLK
    return pl.pallas_call(
        _w_body,
        grid=(n // blk,),
        in_specs=[
            pl.BlockSpec((blk, XPAD), lambda i: (i, 0)),
            pl.BlockSpec((XPAD, FEAT), lambda i: (0, 0)),
        ],
        out_specs=pl.BlockSpec((blk, FEAT), lambda i: (i, 0)),
        out_shape=jax.ShapeDtypeStruct((n, FEAT), jnp.float32),
    )(x_e, waug)


def _dist_features(r, cutoff):
    d = jnp.sqrt(jnp.sum(r * r, axis=-1) + 1e-12)
    n = jnp.arange(1, N_RBF + 1, dtype=jnp.float32)
    rbf = jnp.sin(n[None, :] * (jnp.pi / cutoff) * d[:, None]) / d[:, None]
    env = jnp.where(d < cutoff, 0.5 * (jnp.cos(jnp.pi * d / cutoff) + 1.0), 0.0)
    x = jnp.concatenate([rbf * env[:, None], env[:, None]], axis=1)
    return jnp.pad(x, ((0, 0), (0, XPAD - XDIM)))


def _aug_filter(p):
    w = jnp.concatenate([p['Wd'][:, :FEAT], p['bd'][None, :FEAT]], axis=0)
    return jnp.pad(w, ((0, XPAD - XDIM), (0, 0)))


def kernel(z, xyz, cg_xyz, mapping, nbr_list, cg_nbr_list, embed_table, msg_params, cg_params):
    h = embed_table[z]
    nbr0 = nbr_list[:, 0]
    nbr1 = nbr_list[:, 1]

    r_ij = xyz[nbr1] - xyz[nbr0]
    x_e = _dist_features(r_ij, CUTOFF)
    r_iI = xyz - cg_xyz[mapping]
    x_c = _dist_features(r_iI, CG_CUTOFF)

    nbr0 = jnp.asarray(nbr0, jnp.int32)
    nbr1 = jnp.asarray(nbr1, jnp.int32)
    map_pad = jnp.concatenate([jnp.asarray(mapping, jnp.int32),
                               jnp.full((_APAD - N_ATOMS,), N_CG, jnp.int32)])
    dhp = [None] * N_CONV
    shp = cntp = None
    for i in range(N_CONV):
        p = msg_params[i]
        phi = _phi_call(h, p['W1'], p['b1'][None, :], p['W2'][:, :FEAT], p['b2'][None, :FEAT])
        w_e = _w_call(x_e, _aug_filter(p))
        dsp = _edge_sc_call(phi, w_e, nbr0, nbr1)
        q = cg_params[i]
        h, sc = _contract_call(h, dsp, q['W1'], q['b1'][None, :], q['W2'][:, :FEAT],
                               q['b2'][None, :FEAT], x_c, _aug_filter(q))
        sc_pad = jnp.pad(sc, ((0, _APAD - N_ATOMS), (0, 0)))
        if i == 0:
            h_pad = jnp.pad(h, ((0, _APAD - N_ATOMS), (0, 0)))
            dhp[i], shp, cntp = _contract0_sc_call(sc_pad, h_pad, map_pad)
        else:
            dhp[i] = _contract_sc_call(sc_pad, map_pad)
    H = _assemble_call(shp, cntp, dhp[0], dhp[1], dhp[2])[:N_CG]
    return (H, h)


# speedup vs baseline: 2.6256x; 1.1320x over previous
import functools

import jax
import jax.numpy as jnp
from jax import lax
from jax.experimental import pallas as pl
from jax.experimental.pallas import tpu as pltpu
from jax.experimental.pallas import tpu_sc as plsc

N_ATOMS = 10000
N_CG = 1000
N_EDGES = 160000
FEAT = 128
N_RBF = 20
N_CONV = 3
CUTOFF = 5.0
CG_CUTOFF = 20.0

XDIM = N_RBF + 1
XPAD = 24

_ATOM_BLK = 2000
_EDGE_BLK = 3200


def _swish(x):
    return x * jax.nn.sigmoid(x)


def _phi_body(h_ref, w1_ref, b1_ref, w2_ref, b2_ref, out_ref):
    t = jnp.dot(h_ref[...], w1_ref[...], preferred_element_type=jnp.float32)
    t = t + b1_ref[...]
    t = _swish(t)
    out_ref[...] = jnp.dot(t, w2_ref[...], preferred_element_type=jnp.float32) + b2_ref[...]


@jax.jit
def _phi_call(h, w1, b1, w2, b2):
    n = h.shape[0]
    grid = (n // _ATOM_BLK,)
    return pl.pallas_call(
        _phi_body,
        grid=grid,
        in_specs=[
            pl.BlockSpec((_ATOM_BLK, FEAT), lambda i: (i, 0)),
            pl.BlockSpec((FEAT, FEAT), lambda i: (0, 0)),
            pl.BlockSpec((1, FEAT), lambda i: (0, 0)),
            pl.BlockSpec((FEAT, FEAT), lambda i: (0, 0)),
            pl.BlockSpec((1, FEAT), lambda i: (0, 0)),
        ],
        out_specs=pl.BlockSpec((_ATOM_BLK, FEAT), lambda i: (i, 0)),
        out_shape=jax.ShapeDtypeStruct((n, FEAT), jnp.float32),
    )(h, w1, b1, w2, b2)


def _edge_s0_body(phi_g_ref, x_ref, waug_ref, out_ref):
    w = jnp.dot(x_ref[...], waug_ref[...], preferred_element_type=jnp.float32)
    out_ref[...] = phi_g_ref[...] * w


@jax.jit
def _edge_s0_call(phi_g, x_e, waug):
    n = phi_g.shape[0]
    blk = _EDGE_BLK if n % _EDGE_BLK == 0 else _ATOM_BLK
    grid = (n // blk,)
    return pl.pallas_call(
        _edge_s0_body,
        grid=grid,
        in_specs=[
            pl.BlockSpec((blk, FEAT), lambda i: (i, 0)),
            pl.BlockSpec((blk, XPAD), lambda i: (i, 0)),
            pl.BlockSpec((XPAD, FEAT), lambda i: (0, 0)),
        ],
        out_specs=pl.BlockSpec((blk, FEAT), lambda i: (i, 0)),
        out_shape=jax.ShapeDtypeStruct((n, FEAT), jnp.float32),
    )(phi_g, x_e, waug)


def _contract_body(h_ref, p0_ref, p1_ref, w1_ref, b1_ref, w2_ref, b2_ref,
                   x_ref, waug_ref, hnew_ref, out_ref):
    hn = h_ref[...] + p0_ref[0] + p1_ref[0]
    hnew_ref[...] = hn
    t = jnp.dot(hn, w1_ref[...], preferred_element_type=jnp.float32)
    t = t + b1_ref[...]
    t = _swish(t)
    phi = jnp.dot(t, w2_ref[...], preferred_element_type=jnp.float32) + b2_ref[...]
    w = jnp.dot(x_ref[...], waug_ref[...], preferred_element_type=jnp.float32)
    out_ref[...] = phi * w


@jax.jit
def _contract_call(h, dsp, w1, b1, w2, b2, x_c, waug):
    n = h.shape[0]
    grid = (n // _ATOM_BLK,)
    return pl.pallas_call(
        _contract_body,
        grid=grid,
        in_specs=[
            pl.BlockSpec((_ATOM_BLK, FEAT), lambda i: (i, 0)),
            pl.BlockSpec((1, _ATOM_BLK, FEAT), lambda i: (0, i, 0)),
            pl.BlockSpec((1, _ATOM_BLK, FEAT), lambda i: (1, i, 0)),
            pl.BlockSpec((FEAT, FEAT), lambda i: (0, 0)),
            pl.BlockSpec((1, FEAT), lambda i: (0, 0)),
            pl.BlockSpec((FEAT, FEAT), lambda i: (0, 0)),
            pl.BlockSpec((1, FEAT), lambda i: (0, 0)),
            pl.BlockSpec((_ATOM_BLK, XPAD), lambda i: (i, 0)),
            pl.BlockSpec((XPAD, FEAT), lambda i: (0, 0)),
        ],
        out_specs=[
            pl.BlockSpec((_ATOM_BLK, FEAT), lambda i: (i, 0)),
            pl.BlockSpec((_ATOM_BLK, FEAT), lambda i: (i, 0)),
        ],
        out_shape=[
            jax.ShapeDtypeStruct((n, FEAT), jnp.float32),
            jax.ShapeDtypeStruct((n, FEAT), jnp.float32),
        ],
    )(h, dsp, dsp, w1, b1, w2, b2, x_c, waug)


_NC = 2
_NS = 16
_NW = _NC * _NS
_EC = 64
_NBUF = 2
_ECHUNKS = N_EDGES // _EC
_ROWS_PER_TILE = 624


def _edge_sc_body(phi_hbm, w_hbm, n0_hbm, n1_hbm, out_hbm,
                  acc, idx0_v, idx1_v, sidx_v, w_v, g_v,
                  sem_in0, sem_in1, sem_s0, sem_s1):
    c = lax.axis_index("c")
    s = lax.axis_index("s")
    wid = s * _NC + c

    def _zfill(r, carry):
        for k in range(8):
            w_v[0, r, pl.ds(k * 16, 16)] = jnp.zeros((16,), jnp.float32)
        return carry
    lax.fori_loop(0, _EC, _zfill, 0)
    base_r = s * _ROWS_PER_TILE
    for k in range(9):
        pltpu.sync_copy(w_v.at[0], acc.at[pl.ds(base_r + k * _EC, _EC), :])
    pltpu.sync_copy(w_v.at[0, pl.ds(0, _ROWS_PER_TILE - 9 * _EC), :],
                    acc.at[pl.ds(base_r + 9 * _EC, _ROWS_PER_TILE - 9 * _EC), :])

    @pl.when(s == _NS - 1)
    def _zero_tail():
        pltpu.sync_copy(w_v.at[0, pl.ds(0, N_ATOMS - _NS * _ROWS_PER_TILE), :],
                        acc.at[pl.ds(_NS * _ROWS_PER_TILE, N_ATOMS - _NS * _ROWS_PER_TILE), :])
    plsc.subcore_barrier()

    nch = jnp.where(wid < _ECHUNKS - (_ECHUNKS // _NW) * _NW,
                    _ECHUNKS // _NW + 1, _ECHUNKS // _NW)
    sem_in = (sem_in0, sem_in1)
    sem_s = (sem_s0, sem_s1)

    def _issue_inputs(jj, b):
        base = (jj * _NW + wid) * _EC
        pltpu.async_copy(n0_hbm.at[pl.ds(base, _EC)], idx0_v.at[b], sem_in[b])
        pltpu.async_copy(n1_hbm.at[pl.ds(base, _EC)], idx1_v.at[b], sem_in[b])
        pltpu.async_copy(w_hbm.at[pl.ds(base, _EC), :], w_v.at[b], sem_in[b])

    def _wait_inputs(jj, b):
        base = (jj * _NW + wid) * _EC
        pltpu.make_async_copy(n0_hbm.at[pl.ds(base, _EC)], idx0_v.at[b], sem_in[b]).wait()
        pltpu.make_async_copy(n1_hbm.at[pl.ds(base, _EC)], idx1_v.at[b], sem_in[b]).wait()
        pltpu.make_async_copy(w_hbm.at[pl.ds(base, _EC), :], w_v.at[b], sem_in[b]).wait()

    def _wait_scatter(b):
        pltpu.make_async_copy(g_v.at[b], acc.at[sidx_v.at[b]], sem_s[b]).wait()

    for b in range(_NBUF):
        _issue_inputs(b, b)

    def _step(t, carry):
        for b in range(_NBUF):
            jj = _NBUF * t + b

            @pl.when(jj < nch)
            def _do():
                @pl.when(t > 0)
                def _drain():
                    _wait_scatter(b)
                _wait_inputs(jj, b)
                pltpu.sync_copy(phi_hbm.at[idx1_v.at[b]], g_v.at[b])

                def _mul(r, inner):
                    for k in range(8):
                        sl = pl.ds(k * 16, 16)
                        g_v[b, r, sl] = g_v[b, r, sl] * w_v[b, r, sl]
                    return inner
                lax.fori_loop(0, _EC, _mul, 0)

                def _cpidx(k, inner):
                    sl = pl.ds(k * 16, 16)
                    sidx_v[b, sl] = idx0_v[b, sl]
                    return inner
                lax.fori_loop(0, 8, _cpidx, 0)
                pltpu.async_copy(g_v.at[b], acc.at[sidx_v.at[b]], sem_s[b], add=True)

                @pl.when(jj + _NBUF < nch)
                def _prefetch():
                    _issue_inputs(jj + _NBUF, b)
        return carry
    lax.fori_loop(0, (_ECHUNKS // _NW + _NBUF) // _NBUF, _step, 0)
    for b in range(_NBUF):
        _wait_scatter(b)
    plsc.subcore_barrier()
    pltpu.sync_copy(acc.at[pl.ds(base_r, _ROWS_PER_TILE), :],
                    out_hbm.at[c, pl.ds(base_r, _ROWS_PER_TILE), :])

    @pl.when(s == _NS - 1)
    def _flush_tail():
        pltpu.sync_copy(acc.at[pl.ds(_NS * _ROWS_PER_TILE, N_ATOMS - _NS * _ROWS_PER_TILE), :],
                        out_hbm.at[c, pl.ds(_NS * _ROWS_PER_TILE, N_ATOMS - _NS * _ROWS_PER_TILE), :])


@jax.jit
def _edge_sc_call(phi, w_e, nbr0, nbr1):
    return pl.kernel(
        _edge_sc_body,
        out_type=jax.ShapeDtypeStruct((_NC, N_ATOMS, FEAT), jnp.float32),
        mesh=plsc.VectorSubcoreMesh(core_axis_name="c", subcore_axis_name="s"),
        scratch_types=[
            pltpu.VMEM_SHARED((N_ATOMS, FEAT), jnp.float32),
            pltpu.VMEM((_NBUF, _EC), jnp.int32),
            pltpu.VMEM((_NBUF, _EC), jnp.int32),
            pltpu.VMEM((_NBUF, _EC), jnp.int32),
            pltpu.VMEM((_NBUF, _EC, FEAT), jnp.float32),
            pltpu.VMEM((_NBUF, _EC, FEAT), jnp.float32),
            pltpu.SemaphoreType.DMA,
            pltpu.SemaphoreType.DMA,
            pltpu.SemaphoreType.DMA,
            pltpu.SemaphoreType.DMA,
        ],
    )(phi, w_e, nbr0, nbr1)


_APAD = 10240
_CGPAD = 1024
_CROWS = _APAD // _NW


def _cg_zero_flush(c, s, accs, buf, outs):
    def _zfill(r, carry):
        for k in range(8):
            buf[r, pl.ds(k * 16, 16)] = jnp.zeros((16,), jnp.float32)
        return carry
    lax.fori_loop(0, 64, _zfill, 0)
    for acc in accs:
        w = acc.shape[1]
        pltpu.sync_copy(buf.at[pl.ds(0, 64), pl.ds(0, w)], acc.at[pl.ds(s * 64, 64), :])


def _contract_sc_body(sc_hbm, map_hbm, out_hbm, acc, idxA, idxB, buf):
    c = lax.axis_index("c")
    s = lax.axis_index("s")
    wid = s * _NC + c
    _cg_zero_flush(c, s, [acc], buf, None)
    plsc.subcore_barrier()
    base = wid * _CROWS
    for off in (0, 128):
        pltpu.sync_copy(map_hbm.at[pl.ds(base + off, 128)], idxA)
        pltpu.sync_copy(sc_hbm.at[pl.ds(base + off, 128), :], buf)
        pltpu.sync_copy(buf, acc.at[idxA], add=True)
    pltpu.sync_copy(map_hbm.at[pl.ds(base + 256, 64)], idxB)
    pltpu.sync_copy(sc_hbm.at[pl.ds(base + 256, 64), :], buf.at[pl.ds(0, 64), :])
    pltpu.sync_copy(buf.at[pl.ds(0, 64), :], acc.at[idxB], add=True)
    plsc.subcore_barrier()
    pltpu.sync_copy(acc.at[pl.ds(s * 64, 64), :], out_hbm.at[c, pl.ds(s * 64, 64), :])


@jax.jit
def _contract_sc_call(sc_pad, map_pad):
    return pl.kernel(
        _contract_sc_body,
        out_type=jax.ShapeDtypeStruct((_NC, _CGPAD, FEAT), jnp.float32),
        mesh=plsc.VectorSubcoreMesh(core_axis_name="c", subcore_axis_name="s"),
        scratch_types=[
            pltpu.VMEM_SHARED((_CGPAD, FEAT), jnp.float32),
            pltpu.VMEM((128,), jnp.int32),
            pltpu.VMEM((64,), jnp.int32),
            pltpu.VMEM((128, FEAT), jnp.float32),
        ],
    )(sc_pad, map_pad)


def _contract0_sc_body(sc_hbm, h_hbm, map_hbm, out_hbm, outsh_hbm, outcnt_hbm,
                       acc, accsh, acccnt, idxA, idxB, buf, ones_v):
    c = lax.axis_index("c")
    s = lax.axis_index("s")
    wid = s * _NC + c

    def _ofill(r, carry):
        for k in range(8):
            ones_v[r, pl.ds(k * 16, 16)] = jnp.full((16,), 1.0, jnp.float32)
        return carry
    lax.fori_loop(0, 128, _ofill, 0)
    _cg_zero_flush(c, s, [acc, accsh, acccnt], buf, None)
    plsc.subcore_barrier()
    base = wid * _CROWS
    for off in (0, 128):
        pltpu.sync_copy(map_hbm.at[pl.ds(base + off, 128)], idxA)
        pltpu.sync_copy(sc_hbm.at[pl.ds(base + off, 128), :], buf)
        pltpu.sync_copy(buf, acc.at[idxA], add=True)
        pltpu.sync_copy(h_hbm.at[pl.ds(base + off, 128), :], buf)
        pltpu.sync_copy(buf, accsh.at[idxA], add=True)
        pltpu.sync_copy(ones_v, acccnt.at[idxA], add=True)
    pltpu.sync_copy(map_hbm.at[pl.ds(base + 256, 64)], idxB)
    pltpu.sync_copy(sc_hbm.at[pl.ds(base + 256, 64), :], buf.at[pl.ds(0, 64), :])
    pltpu.sync_copy(buf.at[pl.ds(0, 64), :], acc.at[idxB], add=True)
    pltpu.sync_copy(h_hbm.at[pl.ds(base + 256, 64), :], buf.at[pl.ds(0, 64), :])
    pltpu.sync_copy(buf.at[pl.ds(0, 64), :], accsh.at[idxB], add=True)
    pltpu.sync_copy(ones_v.at[pl.ds(0, 64), :], acccnt.at[idxB], add=True)
    plsc.subcore_barrier()
    pltpu.sync_copy(acc.at[pl.ds(s * 64, 64), :], out_hbm.at[c, pl.ds(s * 64, 64), :])
    pltpu.sync_copy(accsh.at[pl.ds(s * 64, 64), :], outsh_hbm.at[c, pl.ds(s * 64, 64), :])
    pltpu.sync_copy(acccnt.at[pl.ds(s * 64, 64), :], outcnt_hbm.at[c, pl.ds(s * 64, 64), :])


@jax.jit
def _contract0_sc_call(sc_pad, h_pad, map_pad):
    return pl.kernel(
        _contract0_sc_body,
        out_type=(
            jax.ShapeDtypeStruct((_NC, _CGPAD, FEAT), jnp.float32),
            jax.ShapeDtypeStruct((_NC, _CGPAD, FEAT), jnp.float32),
            jax.ShapeDtypeStruct((_NC, _CGPAD, FEAT), jnp.float32),
        ),
        mesh=plsc.VectorSubcoreMesh(core_axis_name="c", subcore_axis_name="s"),
        scratch_types=[
            pltpu.VMEM_SHARED((_CGPAD, FEAT), jnp.float32),
            pltpu.VMEM_SHARED((_CGPAD, FEAT), jnp.float32),
            pltpu.VMEM_SHARED((_CGPAD, FEAT), jnp.float32),
            pltpu.VMEM((128,), jnp.int32),
            pltpu.VMEM((64,), jnp.int32),
            pltpu.VMEM((128, FEAT), jnp.float32),
            pltpu.VMEM((128, FEAT), jnp.float32),
        ],
    )(sc_pad, h_pad, map_pad)


def _assemble_body(sh_ref, cnt_ref, d0_ref, d1_ref, d2_ref, out_ref):
    sh = sh_ref[0] + sh_ref[1]
    cnt = cnt_ref[0][:, :1] + cnt_ref[1][:, :1]
    out_ref[...] = (sh / jnp.clip(cnt, 1.0)
                    + d0_ref[0] + d0_ref[1]
                    + d1_ref[0] + d1_ref[1]
                    + d2_ref[0] + d2_ref[1])


@jax.jit
def _assemble_call(shp, cntp, d0p, d1p, d2p):
    full = lambda shape: pl.BlockSpec(shape, lambda: (0, 0, 0))
    return pl.pallas_call(
        _assemble_body,
        in_specs=[
            full((_NC, _CGPAD, FEAT)),
            full((_NC, _CGPAD, FEAT)),
            full((_NC, _CGPAD, FEAT)),
            full((_NC, _CGPAD, FEAT)),
            full((_NC, _CGPAD, FEAT)),
        ],
        out_specs=pl.BlockSpec((_CGPAD, FEAT), lambda: (0, 0)),
        out_shape=jax.ShapeDtypeStruct((_CGPAD, FEAT), jnp.float32),
    )(shp, cntp, d0p, d1p, d2p)


def _w_body(x_ref, waug_ref, out_ref):
    out_ref[...] = jnp.dot(x_ref[...], waug_ref[...], preferred_element_type=jnp.float32)


@jax.jit
def _w_call(x_e, waug):
    n = x_e.shape[0]
    blk = _EDGE_BLK if n % _EDGE_BLK == 0 else _ATOM_BLK
    return pl.pallas_call(
        _w_body,
        grid=(n // blk,),
        in_specs=[
            pl.BlockSpec((blk, XPAD), lambda i: (i, 0)),
            pl.BlockSpec((XPAD, FEAT), lambda i: (0, 0)),
        ],
        out_specs=pl.BlockSpec((blk, FEAT), lambda i: (i, 0)),
        out_shape=jax.ShapeDtypeStruct((n, FEAT), jnp.float32),
    )(x_e, waug)


def _dist_features(r, cutoff):
    d = jnp.sqrt(jnp.sum(r * r, axis=-1) + 1e-12)
    n = jnp.arange(1, N_RBF + 1, dtype=jnp.float32)
    rbf = jnp.sin(n[None, :] * (jnp.pi / cutoff) * d[:, None]) / d[:, None]
    env = jnp.where(d < cutoff, 0.5 * (jnp.cos(jnp.pi * d / cutoff) + 1.0), 0.0)
    x = jnp.concatenate([rbf * env[:, None], env[:, None]], axis=1)
    return jnp.pad(x, ((0, 0), (0, XPAD - XDIM)))


def _aug_filter(p):
    w = jnp.concatenate([p['Wd'][:, :FEAT], p['bd'][None, :FEAT]], axis=0)
    return jnp.pad(w, ((0, XPAD - XDIM), (0, 0)))


def kernel(z, xyz, cg_xyz, mapping, nbr_list, cg_nbr_list, embed_table, msg_params, cg_params):
    h = embed_table[z]
    nbr0 = nbr_list[:, 0]
    nbr1 = nbr_list[:, 1]

    r_ij = xyz[nbr1] - xyz[nbr0]
    x_e = _dist_features(r_ij, CUTOFF)
    r_iI = xyz - cg_xyz[mapping]
    x_c = _dist_features(r_iI, CG_CUTOFF)

    nbr0 = jnp.asarray(nbr0, jnp.int32)
    nbr1 = jnp.asarray(nbr1, jnp.int32)
    map_pad = jnp.concatenate([jnp.asarray(mapping, jnp.int32),
                               jnp.full((_APAD - N_ATOMS,), N_CG, jnp.int32)])
    dhp = [None] * N_CONV
    shp = cntp = None
    for i in range(N_CONV):
        p = msg_params[i]
        phi = _phi_call(h, p['W1'], p['b1'][None, :], p['W2'][:, :FEAT], p['b2'][None, :FEAT])
        w_e = _w_call(x_e, _aug_filter(p))
        dsp = _edge_sc_call(phi, w_e, nbr0, nbr1)
        q = cg_params[i]
        h, sc = _contract_call(h, dsp, q['W1'], q['b1'][None, :], q['W2'][:, :FEAT],
                               q['b2'][None, :FEAT], x_c, _aug_filter(q))
        sc_pad = jnp.pad(sc, ((0, _APAD - N_ATOMS), (0, 0)))
        if i == 0:
            h_pad = jnp.pad(h, ((0, _APAD - N_ATOMS), (0, 0)))
            dhp[i], shp, cntp = _contract0_sc_call(sc_pad, h_pad, map_pad)
        else:
            dhp[i] = _contract_sc_call(sc_pad, map_pad)
    H = _assemble_call(shp, cntp, dhp[0], dhp[1], dhp[2])[:N_CG]
    return (H, h)


# speedup vs baseline: 5.1110x; 1.9466x over previous
import functools

import jax
import jax.numpy as jnp
from jax import lax
from jax.experimental import pallas as pl
from jax.experimental.pallas import tpu as pltpu
from jax.experimental.pallas import tpu_sc as plsc

N_ATOMS = 10000
N_CG = 1000
N_EDGES = 160000
FEAT = 128
N_RBF = 20
N_CONV = 3
CUTOFF = 5.0
CG_CUTOFF = 20.0

XDIM = N_RBF + 1
XPAD = 24

_ATOM_BLK = 2048


def _swish(x):
    return x * jax.nn.sigmoid(x)


def _phi_body(h_ref, w1_ref, b1_ref, w2_ref, b2_ref, out_ref):
    t = jnp.dot(h_ref[...], w1_ref[...], preferred_element_type=jnp.float32)
    t = t + b1_ref[...]
    t = _swish(t)
    out_ref[...] = jnp.dot(t, w2_ref[...], preferred_element_type=jnp.float32) + b2_ref[...]


@jax.jit
def _phi_call(h, w1, b1, w2, b2):
    n = h.shape[0]
    grid = (n // _ATOM_BLK,)
    return pl.pallas_call(
        _phi_body,
        grid=grid,
        in_specs=[
            pl.BlockSpec((_ATOM_BLK, FEAT), lambda i: (i, 0)),
            pl.BlockSpec((FEAT, FEAT), lambda i: (0, 0)),
            pl.BlockSpec((1, FEAT), lambda i: (0, 0)),
            pl.BlockSpec((FEAT, FEAT), lambda i: (0, 0)),
            pl.BlockSpec((1, FEAT), lambda i: (0, 0)),
        ],
        out_specs=pl.BlockSpec((_ATOM_BLK, FEAT), lambda i: (i, 0)),
        out_shape=jax.ShapeDtypeStruct((n, FEAT), jnp.float32),
    )(h, w1, b1, w2, b2)


def _contract_body(h_ref, p0_ref, p1_ref, w1_ref, b1_ref, w2_ref, b2_ref,
                   xt_ref, waug_ref, hnew_ref, out_ref):
    hn = h_ref[...] + p0_ref[0] + p1_ref[0]
    hnew_ref[...] = hn
    t = jnp.dot(hn, w1_ref[...], preferred_element_type=jnp.float32)
    t = t + b1_ref[...]
    t = _swish(t)
    phi = jnp.dot(t, w2_ref[...], preferred_element_type=jnp.float32) + b2_ref[...]
    w = lax.dot_general(xt_ref[...], waug_ref[...], (((0,), (0,)), ((), ())),
                        preferred_element_type=jnp.float32)
    out_ref[...] = phi * w


@jax.jit
def _contract_call(h, dsp, w1, b1, w2, b2, xt_c, waug):
    n = h.shape[0]
    grid = (n // _ATOM_BLK,)
    return pl.pallas_call(
        _contract_body,
        grid=grid,
        in_specs=[
            pl.BlockSpec((_ATOM_BLK, FEAT), lambda i: (i, 0)),
            pl.BlockSpec((1, _ATOM_BLK, FEAT), lambda i: (0, i, 0)),
            pl.BlockSpec((1, _ATOM_BLK, FEAT), lambda i: (1, i, 0)),
            pl.BlockSpec((FEAT, FEAT), lambda i: (0, 0)),
            pl.BlockSpec((1, FEAT), lambda i: (0, 0)),
            pl.BlockSpec((FEAT, FEAT), lambda i: (0, 0)),
            pl.BlockSpec((1, FEAT), lambda i: (0, 0)),
            pl.BlockSpec((XPAD, _ATOM_BLK), lambda i: (0, i)),
            pl.BlockSpec((XPAD, FEAT), lambda i: (0, 0)),
        ],
        out_specs=[
            pl.BlockSpec((_ATOM_BLK, FEAT), lambda i: (i, 0)),
            pl.BlockSpec((_ATOM_BLK, FEAT), lambda i: (i, 0)),
        ],
        out_shape=[
            jax.ShapeDtypeStruct((n, FEAT), jnp.float32),
            jax.ShapeDtypeStruct((n, FEAT), jnp.float32),
        ],
    )(h, dsp, dsp, w1, b1, w2, b2, xt_c, waug)


_NC = 2
_NS = 16
_NW = _NC * _NS
_EC = 64
_NBUF = 2
_ECHUNKS = N_EDGES // _EC
_ROWS_PER_TILE = 10240 // _NS


def _edge_sc_body(phi_hbm, w_hbm, n0_hbm, n1_hbm, out_hbm,
                  acc, idx0_v, idx1_v, sidx_v, w_v, g_v,
                  sem_in0, sem_in1, sem_s0, sem_s1):
    c = lax.axis_index("c")
    s = lax.axis_index("s")
    wid = s * _NC + c

    def _zfill(r, carry):
        for k in range(8):
            w_v[0, r, pl.ds(k * 16, 16)] = jnp.zeros((16,), jnp.float32)
        return carry
    lax.fori_loop(0, _EC, _zfill, 0)
    base_r = s * _ROWS_PER_TILE
    for k in range(_ROWS_PER_TILE // _EC):
        pltpu.sync_copy(w_v.at[0], acc.at[pl.ds(base_r + k * _EC, _EC), :])
    plsc.subcore_barrier()

    nch = jnp.where(wid < _ECHUNKS - (_ECHUNKS // _NW) * _NW,
                    _ECHUNKS // _NW + 1, _ECHUNKS // _NW)
    sem_in = (sem_in0, sem_in1)
    sem_s = (sem_s0, sem_s1)

    def _issue_inputs(jj, b):
        base = (jj * _NW + wid) * _EC
        pltpu.async_copy(n0_hbm.at[pl.ds(base, _EC)], idx0_v.at[b], sem_in[b])
        pltpu.async_copy(n1_hbm.at[pl.ds(base, _EC)], idx1_v.at[b], sem_in[b])
        pltpu.async_copy(w_hbm.at[pl.ds(base, _EC), :], w_v.at[b], sem_in[b])

    def _wait_inputs(jj, b):
        base = (jj * _NW + wid) * _EC
        pltpu.make_async_copy(n0_hbm.at[pl.ds(base, _EC)], idx0_v.at[b], sem_in[b]).wait()
        pltpu.make_async_copy(n1_hbm.at[pl.ds(base, _EC)], idx1_v.at[b], sem_in[b]).wait()
        pltpu.make_async_copy(w_hbm.at[pl.ds(base, _EC), :], w_v.at[b], sem_in[b]).wait()

    def _wait_scatter(b):
        pltpu.make_async_copy(g_v.at[b], acc.at[sidx_v.at[b]], sem_s[b]).wait()

    for b in range(_NBUF):
        _issue_inputs(b, b)

    def _step(t, carry):
        for b in range(_NBUF):
            jj = _NBUF * t + b

            @pl.when(jj < nch)
            def _do():
                @pl.when(t > 0)
                def _drain():
                    _wait_scatter(b)
                _wait_inputs(jj, b)
                pltpu.sync_copy(phi_hbm.at[idx1_v.at[b]], g_v.at[b])

                def _mul(r, inner):
                    for k in range(8):
                        sl = pl.ds(k * 16, 16)
                        g_v[b, r, sl] = g_v[b, r, sl] * w_v[b, r, sl]
                    return inner
                lax.fori_loop(0, _EC, _mul, 0)

                def _cpidx(k, inner):
                    sl = pl.ds(k * 16, 16)
                    sidx_v[b, sl] = idx0_v[b, sl]
                    return inner
                lax.fori_loop(0, 8, _cpidx, 0)
                pltpu.async_copy(g_v.at[b], acc.at[sidx_v.at[b]], sem_s[b], add=True)

                @pl.when(jj + _NBUF < nch)
                def _prefetch():
                    _issue_inputs(jj + _NBUF, b)
        return carry
    lax.fori_loop(0, (_ECHUNKS // _NW + _NBUF) // _NBUF, _step, 0)
    for b in range(_NBUF):
        _wait_scatter(b)
    plsc.subcore_barrier()
    pltpu.sync_copy(acc.at[pl.ds(base_r, _ROWS_PER_TILE), :],
                    out_hbm.at[c, pl.ds(base_r, _ROWS_PER_TILE), :])


@jax.jit
def _edge_sc_call(phi, w_e, nbr0, nbr1):
    return pl.kernel(
        _edge_sc_body,
        out_type=jax.ShapeDtypeStruct((_NC, 10240, FEAT), jnp.float32),
        mesh=plsc.VectorSubcoreMesh(core_axis_name="c", subcore_axis_name="s"),
        scratch_types=[
            pltpu.VMEM_SHARED((10240, FEAT), jnp.float32),
            pltpu.VMEM((_NBUF, _EC), jnp.int32),
            pltpu.VMEM((_NBUF, _EC), jnp.int32),
            pltpu.VMEM((_NBUF, _EC), jnp.int32),
            pltpu.VMEM((_NBUF, _EC, FEAT), jnp.float32),
            pltpu.VMEM((_NBUF, _EC, FEAT), jnp.float32),
            pltpu.SemaphoreType.DMA,
            pltpu.SemaphoreType.DMA,
            pltpu.SemaphoreType.DMA,
            pltpu.SemaphoreType.DMA,
        ],
    )(phi, w_e, nbr0, nbr1)


_APAD = 10240
_CGPAD = 1024
_CROWS = _APAD // _NW


def _cg_zero_flush(c, s, accs, buf, outs):
    def _zfill(r, carry):
        for k in range(8):
            buf[r, pl.ds(k * 16, 16)] = jnp.zeros((16,), jnp.float32)
        return carry
    lax.fori_loop(0, 64, _zfill, 0)
    for acc in accs:
        w = acc.shape[1]
        pltpu.sync_copy(buf.at[pl.ds(0, 64), pl.ds(0, w)], acc.at[pl.ds(s * 64, 64), :])


def _contract_sc_body(sc_hbm, map_hbm, out_hbm, acc, idxA, idxB, buf):
    c = lax.axis_index("c")
    s = lax.axis_index("s")
    wid = s * _NC + c
    _cg_zero_flush(c, s, [acc], buf, None)
    plsc.subcore_barrier()
    base = wid * _CROWS
    for off in (0, 128):
        pltpu.sync_copy(map_hbm.at[pl.ds(base + off, 128)], idxA)
        pltpu.sync_copy(sc_hbm.at[pl.ds(base + off, 128), :], buf)
        pltpu.sync_copy(buf, acc.at[idxA], add=True)
    pltpu.sync_copy(map_hbm.at[pl.ds(base + 256, 64)], idxB)
    pltpu.sync_copy(sc_hbm.at[pl.ds(base + 256, 64), :], buf.at[pl.ds(0, 64), :])
    pltpu.sync_copy(buf.at[pl.ds(0, 64), :], acc.at[idxB], add=True)
    plsc.subcore_barrier()
    pltpu.sync_copy(acc.at[pl.ds(s * 64, 64), :], out_hbm.at[c, pl.ds(s * 64, 64), :])


@jax.jit
def _contract_sc_call(sc_pad, map_pad):
    return pl.kernel(
        _contract_sc_body,
        out_type=jax.ShapeDtypeStruct((_NC, _CGPAD, FEAT), jnp.float32),
        mesh=plsc.VectorSubcoreMesh(core_axis_name="c", subcore_axis_name="s"),
        scratch_types=[
            pltpu.VMEM_SHARED((_CGPAD, FEAT), jnp.float32),
            pltpu.VMEM((128,), jnp.int32),
            pltpu.VMEM((64,), jnp.int32),
            pltpu.VMEM((128, FEAT), jnp.float32),
        ],
    )(sc_pad, map_pad)


def _contract0_sc_body(sc_hbm, h_hbm, map_hbm, out_hbm, outsh_hbm, outcnt_hbm,
                       acc, accsh, acccnt, idxA, idxB, buf, ones_v):
    c = lax.axis_index("c")
    s = lax.axis_index("s")
    wid = s * _NC + c

    def _ofill(r, carry):
        for k in range(8):
            ones_v[r, pl.ds(k * 16, 16)] = jnp.full((16,), 1.0, jnp.float32)
        return carry
    lax.fori_loop(0, 128, _ofill, 0)
    _cg_zero_flush(c, s, [acc, accsh, acccnt], buf, None)
    plsc.subcore_barrier()
    base = wid * _CROWS
    for off in (0, 128):
        pltpu.sync_copy(map_hbm.at[pl.ds(base + off, 128)], idxA)
        pltpu.sync_copy(sc_hbm.at[pl.ds(base + off, 128), :], buf)
        pltpu.sync_copy(buf, acc.at[idxA], add=True)
        pltpu.sync_copy(h_hbm.at[pl.ds(base + off, 128), :], buf)
        pltpu.sync_copy(buf, accsh.at[idxA], add=True)
        pltpu.sync_copy(ones_v, acccnt.at[idxA], add=True)
    pltpu.sync_copy(map_hbm.at[pl.ds(base + 256, 64)], idxB)
    pltpu.sync_copy(sc_hbm.at[pl.ds(base + 256, 64), :], buf.at[pl.ds(0, 64), :])
    pltpu.sync_copy(buf.at[pl.ds(0, 64), :], acc.at[idxB], add=True)
    pltpu.sync_copy(h_hbm.at[pl.ds(base + 256, 64), :], buf.at[pl.ds(0, 64), :])
    pltpu.sync_copy(buf.at[pl.ds(0, 64), :], accsh.at[idxB], add=True)
    pltpu.sync_copy(ones_v.at[pl.ds(0, 64), :], acccnt.at[idxB], add=True)
    plsc.subcore_barrier()
    pltpu.sync_copy(acc.at[pl.ds(s * 64, 64), :], out_hbm.at[c, pl.ds(s * 64, 64), :])
    pltpu.sync_copy(accsh.at[pl.ds(s * 64, 64), :], outsh_hbm.at[c, pl.ds(s * 64, 64), :])
    pltpu.sync_copy(acccnt.at[pl.ds(s * 64, 64), :], outcnt_hbm.at[c, pl.ds(s * 64, 64), :])


@jax.jit
def _contract0_sc_call(sc_pad, h_pad, map_pad):
    return pl.kernel(
        _contract0_sc_body,
        out_type=(
            jax.ShapeDtypeStruct((_NC, _CGPAD, FEAT), jnp.float32),
            jax.ShapeDtypeStruct((_NC, _CGPAD, FEAT), jnp.float32),
            jax.ShapeDtypeStruct((_NC, _CGPAD, FEAT), jnp.float32),
        ),
        mesh=plsc.VectorSubcoreMesh(core_axis_name="c", subcore_axis_name="s"),
        scratch_types=[
            pltpu.VMEM_SHARED((_CGPAD, FEAT), jnp.float32),
            pltpu.VMEM_SHARED((_CGPAD, FEAT), jnp.float32),
            pltpu.VMEM_SHARED((_CGPAD, FEAT), jnp.float32),
            pltpu.VMEM((128,), jnp.int32),
            pltpu.VMEM((64,), jnp.int32),
            pltpu.VMEM((128, FEAT), jnp.float32),
            pltpu.VMEM((128, FEAT), jnp.float32),
        ],
    )(sc_pad, h_pad, map_pad)


def _assemble_body(sh_ref, cnt_ref, d0_ref, d1_ref, d2_ref, out_ref):
    sh = sh_ref[0] + sh_ref[1]
    cnt = cnt_ref[0][:, :1] + cnt_ref[1][:, :1]
    out_ref[...] = (sh / jnp.clip(cnt, 1.0)
                    + d0_ref[0] + d0_ref[1]
                    + d1_ref[0] + d1_ref[1]
                    + d2_ref[0] + d2_ref[1])


@jax.jit
def _assemble_call(shp, cntp, d0p, d1p, d2p):
    full = lambda shape: pl.BlockSpec(shape, lambda: (0, 0, 0))
    return pl.pallas_call(
        _assemble_body,
        in_specs=[
            full((_NC, _CGPAD, FEAT)),
            full((_NC, _CGPAD, FEAT)),
            full((_NC, _CGPAD, FEAT)),
            full((_NC, _CGPAD, FEAT)),
            full((_NC, _CGPAD, FEAT)),
        ],
        out_specs=pl.BlockSpec((_CGPAD, FEAT), lambda: (0, 0)),
        out_shape=jax.ShapeDtypeStruct((_CGPAD, FEAT), jnp.float32),
    )(shp, cntp, d0p, d1p, d2p)


def _w_body(xt_ref, waug_ref, out_ref):
    out_ref[...] = lax.dot_general(xt_ref[...], waug_ref[...], (((0,), (0,)), ((), ())),
                                   preferred_element_type=jnp.float32)


@jax.jit
def _w_call(xt_e, waug):
    n = xt_e.shape[1]
    blk = 4096 if n % 4096 == 0 else 2048
    return pl.pallas_call(
        _w_body,
        grid=(n // blk,),
        in_specs=[
            pl.BlockSpec((XPAD, blk), lambda i: (0, i)),
            pl.BlockSpec((XPAD, FEAT), lambda i: (0, 0)),
        ],
        out_specs=pl.BlockSpec((blk, FEAT), lambda i: (i, 0)),
        out_shape=jax.ShapeDtypeStruct((n, FEAT), jnp.float32),
    )(xt_e, waug)


def _distT_body(cutoff, d2_ref, out_ref):
    blk = out_ref.shape[1]
    d = jnp.sqrt(d2_ref[...].reshape(1, blk) + 1e-12)
    k = lax.broadcasted_iota(jnp.int32, (XPAD, blk), 0)
    kf = k.astype(jnp.float32) + 1.0
    rbf = jnp.sin(kf * (jnp.pi / cutoff) * d) / d
    env = jnp.where(d < cutoff, 0.5 * (jnp.cos(jnp.pi * d / cutoff) + 1.0), 0.0)
    x = jnp.where(k < N_RBF, rbf * env, 0.0)
    x = jnp.where(k == N_RBF, env, x)
    out_ref[...] = x


@functools.partial(jax.jit, static_argnames=("cutoff",))
def _distT_call(d2, cutoff):
    n = d2.shape[0]
    blk = 4096 if n % 4096 == 0 else 2048
    return pl.pallas_call(
        functools.partial(_distT_body, cutoff),
        grid=(n // blk,),
        in_specs=[pl.BlockSpec((blk,), lambda i: (i,))],
        out_specs=pl.BlockSpec((XPAD, blk), lambda i: (0, i)),
        out_shape=jax.ShapeDtypeStruct((XPAD, n), jnp.float32),
    )(d2)


_EPAD = 163840
_GC = 64
_GCHUNKS = N_EDGES // _GC
_CCHUNKS = _APAD // _GC


def _geom_sc_body(xs_hbm, ys_hbm, zs_hbm, cx_hbm, cy_hbm, cz_hbm,
                  n0_hbm, n1_hbm, map_hbm, d2e_hbm, d2c_hbm,
                  xs_v, ys_v, zs_v, cxs_v, cys_v, czs_v, i0_v, i1_v, d2_v):
    c = lax.axis_index("c")
    s = lax.axis_index("s")
    wid = s * _NC + c
    pltpu.sync_copy(xs_hbm, xs_v)
    pltpu.sync_copy(ys_hbm, ys_v)
    pltpu.sync_copy(zs_hbm, zs_v)
    pltpu.sync_copy(cx_hbm, cxs_v)
    pltpu.sync_copy(cy_hbm, cys_v)
    pltpu.sync_copy(cz_hbm, czs_v)

    nch = jnp.where(wid < _GCHUNKS - (_GCHUNKS // _NW) * _NW,
                    _GCHUNKS // _NW + 1, _GCHUNKS // _NW)

    def _echunk(j, carry):
        base = (j * _NW + wid) * _GC
        pltpu.sync_copy(n0_hbm.at[pl.ds(base, _GC)], i0_v)
        pltpu.sync_copy(n1_hbm.at[pl.ds(base, _GC)], i1_v)
        for k in range(_GC // 16):
            sl = pl.ds(k * 16, 16)
            i0 = i0_v[sl]
            i1 = i1_v[sl]
            dx = plsc.load_gather(xs_v, [i1]) - plsc.load_gather(xs_v, [i0])
            dy = plsc.load_gather(ys_v, [i1]) - plsc.load_gather(ys_v, [i0])
            dz = plsc.load_gather(zs_v, [i1]) - plsc.load_gather(zs_v, [i0])
            d2_v[sl] = dx * dx + dy * dy + dz * dz
        pltpu.sync_copy(d2_v, d2e_hbm.at[pl.ds(base, _GC)])
        return carry
    lax.fori_loop(0, nch, _echunk, 0)

    def _cchunk(j, carry):
        base = (j * _NW + wid) * _GC
        pltpu.sync_copy(map_hbm.at[pl.ds(base, _GC)], i0_v)
        for k in range(_GC // 16):
            sl = pl.ds(k * 16, 16)
            im = i0_v[sl]
            xi = xs_v[pl.ds(base + k * 16, 16)]
            yi = ys_v[pl.ds(base + k * 16, 16)]
            zi = zs_v[pl.ds(base + k * 16, 16)]
            dx = xi - plsc.load_gather(cxs_v, [im])
            dy = yi - plsc.load_gather(cys_v, [im])
            dz = zi - plsc.load_gather(czs_v, [im])
            d2_v[sl] = dx * dx + dy * dy + dz * dz
        pltpu.sync_copy(d2_v, d2c_hbm.at[pl.ds(base, _GC)])
        return carry
    lax.fori_loop(0, _CCHUNKS // _NW, _cchunk, 0)


@jax.jit
def _geom_sc_call(xyzt, cgt, nbr0, nbr1, map_pad):
    coords = tuple(xyzt[k] for k in range(3)) + tuple(cgt[k] for k in range(3))
    return pl.kernel(
        _geom_sc_body,
        out_type=(
            jax.ShapeDtypeStruct((_EPAD,), jnp.float32),
            jax.ShapeDtypeStruct((_APAD,), jnp.float32),
        ),
        mesh=plsc.VectorSubcoreMesh(core_axis_name="c", subcore_axis_name="s"),
        compiler_params=pltpu.CompilerParams(needs_layout_passes=False),
        scratch_types=[
            pltpu.VMEM((_APAD,), jnp.float32),
            pltpu.VMEM((_APAD,), jnp.float32),
            pltpu.VMEM((_APAD,), jnp.float32),
            pltpu.VMEM((_CGPAD,), jnp.float32),
            pltpu.VMEM((_CGPAD,), jnp.float32),
            pltpu.VMEM((_CGPAD,), jnp.float32),
            pltpu.VMEM((_GC,), jnp.int32),
            pltpu.VMEM((_GC,), jnp.int32),
            pltpu.VMEM((_GC,), jnp.float32),
        ],
    )(*coords, nbr0, nbr1, map_pad)


def _aug_filter(p):
    w = jnp.concatenate([p['Wd'][:, :FEAT], p['bd'][None, :FEAT]], axis=0)
    return jnp.pad(w, ((0, XPAD - XDIM), (0, 0)))


def kernel(z, xyz, cg_xyz, mapping, nbr_list, cg_nbr_list, embed_table, msg_params, cg_params):
    h = jnp.pad(embed_table[z], ((0, _APAD - N_ATOMS), (0, 0)))
    nbr0 = jnp.asarray(nbr_list[:, 0], jnp.int32)
    nbr1 = jnp.asarray(nbr_list[:, 1], jnp.int32)
    map_pad = jnp.concatenate([jnp.asarray(mapping, jnp.int32),
                               jnp.full((_APAD - N_ATOMS,), N_CG, jnp.int32)])

    xyzt = jnp.pad(jnp.asarray(xyz, jnp.float32), ((0, _APAD - N_ATOMS), (0, 0))).T
    cgt = jnp.pad(jnp.asarray(cg_xyz, jnp.float32), ((0, _CGPAD - N_CG), (0, 0))).T
    d2e, d2c_pad = _geom_sc_call(xyzt, cgt, nbr0, nbr1, map_pad)
    xt_e = _distT_call(d2e, CUTOFF)
    xt_c = _distT_call(d2c_pad, CG_CUTOFF)[:, :N_ATOMS]
    dhp = [None] * N_CONV
    shp = cntp = None
    for i in range(N_CONV):
        p = msg_params[i]
        phi = _phi_call(h, p['W1'], p['b1'][None, :], p['W2'][:, :FEAT], p['b2'][None, :FEAT])
        w_e = _w_call(xt_e, _aug_filter(p))
        dsp = _edge_sc_call(phi, w_e, nbr0, nbr1)
        q = cg_params[i]
        h, sc = _contract_call(h, dsp, q['W1'], q['b1'][None, :], q['W2'][:, :FEAT],
                               q['b2'][None, :FEAT], xt_c, _aug_filter(q))
        if i == 0:
            dhp[i], shp, cntp = _contract0_sc_call(sc, h, map_pad)
        else:
            dhp[i] = _contract_sc_call(sc, map_pad)
    H = _assemble_call(shp, cntp, dhp[0], dhp[1], dhp[2])[:N_CG]
    return (H, h[:N_ATOMS])
